# baseline (speedup 1.0000x reference)
import functools

import jax
import jax.numpy as jnp
from jax import lax
from jax.experimental import pallas as pl
from jax.experimental.pallas import tpu as pltpu
from jax.experimental.pallas import tpu_sc as plsc

N_TOKENS = 2048
D_MODEL = 768
N_EXPERTS = 50
P_DIM = 256

TOK_TILE = 256
BR = 32
NBLK = 120
PAD_ROWS = NBLK * BR

_SC_CORES = 2
_SC_SUBCORES = 16
_SC_WORKERS = _SC_CORES * _SC_SUBCORES


def _router_body(x_ref, wr_ref, br_ref, wp_ref, bp_ref,
                 idx_ref, probs_ref, params_ref):
    xb = x_ref[...]
    logits = jnp.dot(xb, wr_ref[...], preferred_element_type=jnp.float32)
    logits = logits + br_ref[...]
    probs = jax.nn.softmax(logits, axis=-1)
    probs_ref[...] = probs
    m = jnp.max(probs, axis=-1, keepdims=True)
    ii = lax.broadcasted_iota(jnp.int32, probs.shape, 1)
    idx = jnp.min(jnp.where(probs == m, ii, N_EXPERTS), axis=-1)
    idx_ref[0, 0, :] = idx.astype(jnp.int32)
    params_ref[...] = (
        jnp.dot(xb, wp_ref[...], preferred_element_type=jnp.float32)
        + bp_ref[...]
    )


def _bookkeep_body(idx_ref, inv_ref, be_ref, cumblk_ref):
    idxc = idx_ref[...]
    eid = lax.broadcasted_iota(jnp.int32, (N_TOKENS, N_EXPERTS), 1)
    onehot = (idxc == eid).astype(jnp.float32)
    tril = (lax.broadcasted_iota(jnp.int32, (N_TOKENS, N_TOKENS), 0)
            >= lax.broadcasted_iota(jnp.int32, (N_TOKENS, N_TOKENS), 1)
            ).astype(jnp.float32)
    csum = jnp.dot(tril, onehot,
                   preferred_element_type=jnp.float32).astype(jnp.int32)
    onehot = onehot.astype(jnp.int32)
    rank = jnp.sum(onehot * csum, axis=1) - 1
    counts = csum[N_TOKENS - 1:N_TOKENS, :]
    nblocks = (counts + BR - 1) // BR
    triu = (lax.broadcasted_iota(jnp.int32, (N_EXPERTS, N_EXPERTS), 0)
            <= lax.broadcasted_iota(jnp.int32, (N_EXPERTS, N_EXPERTS), 1)
            ).astype(jnp.float32)
    cumblk = jnp.dot(nblocks.astype(jnp.float32), triu,
                     preferred_element_type=jnp.float32).astype(jnp.int32)
    padded_off = (cumblk - nblocks) * BR
    inv = jnp.sum(onehot * padded_off, axis=1) + rank
    inv_ref[...] = inv[:, None]
    bid = lax.broadcasted_iota(jnp.int32, (NBLK, N_EXPERTS), 0)
    be = jnp.sum((bid >= cumblk).astype(jnp.int32), axis=1)
    be_ref[...] = jnp.minimum(be, N_EXPERTS - 1)[None, :]
    cumblk_ref[...] = cumblk


def _gmm_body(be_ref, nb_ref, x_ref, w_ref, b_ref, o_ref):
    i = pl.program_id(0)

    @pl.when(i < nb_ref[0])
    def _():
        o_ref[...] = (
            jnp.dot(x_ref[...], w_ref[0], preferred_element_type=jnp.float32)
            + b_ref[0]
        )


def _sc_row_scatter(rows, dst_idx, n_out_rows):
    b, d = rows.shape
    rows_per_w = b // _SC_WORKERS
    mesh = plsc.VectorSubcoreMesh(
        core_axis_name="c", subcore_axis_name="s",
        num_cores=_SC_CORES, num_subcores=_SC_SUBCORES)

    @functools.partial(
        pl.kernel, mesh=mesh,
        out_type=jax.ShapeDtypeStruct((n_out_rows, d), jnp.float32),
        scratch_types=[
            pltpu.VMEM((rows_per_w,), jnp.int32),
            pltpu.VMEM((rows_per_w, d), jnp.float32),
            pltpu.SemaphoreType.DMA,
        ],
    )
    def k(rows_hbm, idx_hbm, out_hbm, idx_v, rows_v, sem):
        wid = lax.axis_index("s") * _SC_CORES + lax.axis_index("c")
        base = wid * rows_per_w
        pltpu.sync_copy(idx_hbm.at[pl.ds(base, rows_per_w)], idx_v)
        pltpu.sync_copy(rows_hbm.at[pl.ds(base, rows_per_w)], rows_v)
        pltpu.async_copy(rows_v, out_hbm.at[idx_v], sem).wait()

    return k(rows, dst_idx)


def _sc_row_gather(table, idx, n_rows, chunk):
    _, d = table.shape
    rows_per_w = n_rows // _SC_WORKERS
    n_chunks = rows_per_w // chunk
    mesh = plsc.VectorSubcoreMesh(
        core_axis_name="c", subcore_axis_name="s",
        num_cores=_SC_CORES, num_subcores=_SC_SUBCORES)

    @functools.partial(
        pl.kernel, mesh=mesh,
        out_type=jax.ShapeDtypeStruct((n_rows, d), jnp.float32),
        scratch_types=[
            pltpu.VMEM((rows_per_w,), jnp.int32),
            pltpu.VMEM((chunk, d), jnp.float32),
            pltpu.SemaphoreType.DMA,
        ],
    )
    def k(table_hbm, idx_hbm, out_hbm, idx_v, rows_v, sem):
        wid = lax.axis_index("s") * _SC_CORES + lax.axis_index("c")
        base = wid * rows_per_w
        pltpu.sync_copy(idx_hbm.at[pl.ds(base, rows_per_w)], idx_v)
        for c in range(n_chunks):
            pltpu.async_copy(
                table_hbm.at[idx_v.at[pl.ds(c * chunk, chunk)]],
                rows_v, sem).wait()
            pltpu.sync_copy(rows_v, out_hbm.at[pl.ds(base + c * chunk, chunk)])

    return k(table, idx)


def kernel(x, W_router, b_router, W_adapt, b_adapt, W_param, b_param):
    n_tiles = N_TOKENS // TOK_TILE
    idx3, tool_probs, params = pl.pallas_call(
        _router_body,
        grid=(n_tiles,),
        in_specs=[
            pl.BlockSpec((TOK_TILE, D_MODEL), lambda i: (i, 0)),
            pl.BlockSpec((D_MODEL, N_EXPERTS), lambda i: (0, 0)),
            pl.BlockSpec((1, N_EXPERTS), lambda i: (0, 0)),
            pl.BlockSpec((D_MODEL, P_DIM), lambda i: (0, 0)),
            pl.BlockSpec((1, P_DIM), lambda i: (0, 0)),
        ],
        out_specs=[
            pl.BlockSpec((1, 1, TOK_TILE), lambda i: (i, 0, 0)),
            pl.BlockSpec((TOK_TILE, N_EXPERTS), lambda i: (i, 0)),
            pl.BlockSpec((TOK_TILE, P_DIM), lambda i: (i, 0)),
        ],
        out_shape=[
            jax.ShapeDtypeStruct((n_tiles, 1, TOK_TILE), jnp.int32),
            jax.ShapeDtypeStruct((N_TOKENS, N_EXPERTS), jnp.float32),
            jax.ShapeDtypeStruct((N_TOKENS, P_DIM), jnp.float32),
        ],
    )(x, W_router, b_router.reshape(1, N_EXPERTS),
      W_param, b_param.reshape(1, P_DIM))
    tool_idx = idx3.reshape(N_TOKENS)

    inv2, be2, cumblk2 = pl.pallas_call(
        _bookkeep_body,
        out_shape=[
            jax.ShapeDtypeStruct((N_TOKENS, 1), jnp.int32),
            jax.ShapeDtypeStruct((1, NBLK), jnp.int32),
            jax.ShapeDtypeStruct((1, N_EXPERTS), jnp.int32),
        ],
    )(tool_idx.reshape(N_TOKENS, 1))
    inv_pos = inv2.reshape(N_TOKENS)
    block_expert = be2.reshape(NBLK)
    nblk_live = cumblk2[0, N_EXPERTS - 1:N_EXPERTS]

    x_sorted = _sc_row_scatter(x, inv_pos, PAD_ROWS)

    y_sorted = pl.pallas_call(
        _gmm_body,
        grid_spec=pltpu.PrefetchScalarGridSpec(
            num_scalar_prefetch=2,
            grid=(NBLK,),
            in_specs=[
                pl.BlockSpec((BR, D_MODEL), lambda i, be, nb: (i, 0)),
                pl.BlockSpec((1, D_MODEL, D_MODEL),
                             lambda i, be, nb: (be[i], 0, 0)),
                pl.BlockSpec((1, 1, D_MODEL), lambda i, be, nb: (be[i], 0, 0)),
            ],
            out_specs=pl.BlockSpec((BR, D_MODEL), lambda i, be, nb: (i, 0)),
        ),
        out_shape=jax.ShapeDtypeStruct((PAD_ROWS, D_MODEL), jnp.float32),
    )(block_expert, nblk_live, x_sorted, W_adapt,
      b_adapt.reshape(N_EXPERTS, 1, D_MODEL))

    adapted = _sc_row_gather(y_sorted, inv_pos, N_TOKENS, chunk=64)

    return (tool_idx, tool_probs, adapted, params)


# speedup vs baseline: 1.3807x; 1.3807x over previous
import functools

import jax
import jax.numpy as jnp
from jax import lax
from jax.experimental import pallas as pl
from jax.experimental.pallas import tpu as pltpu
from jax.experimental.pallas import tpu_sc as plsc

N_TOKENS = 2048
D_MODEL = 768
N_EXPERTS = 50
P_DIM = 256

TOK_TILE = 256
BR = 64
NBLK = 84
PAD_ROWS = NBLK * BR

_SC_CORES = 2
_SC_SUBCORES = 16
_SC_WORKERS = _SC_CORES * _SC_SUBCORES


def _router_body(x_ref, wr_ref, br_ref, wp_ref, bp_ref,
                 idx_ref, probs_ref, params_ref):
    xb = x_ref[...]
    logits = jnp.dot(xb, wr_ref[...], preferred_element_type=jnp.float32)
    logits = logits + br_ref[...]
    probs = jax.nn.softmax(logits, axis=-1)
    probs_ref[...] = probs
    m = jnp.max(probs, axis=-1, keepdims=True)
    ii = lax.broadcasted_iota(jnp.int32, probs.shape, 1)
    idx = jnp.min(jnp.where(probs == m, ii, N_EXPERTS), axis=-1)
    idx_ref[0, 0, :] = idx.astype(jnp.int32)
    params_ref[...] = (
        jnp.dot(xb, wp_ref[...], preferred_element_type=jnp.float32)
        + bp_ref[...]
    )


def _bookkeep_body(idx_ref, inv_ref, be_ref, cumblk_ref):
    idxc = idx_ref[...]
    eid = lax.broadcasted_iota(jnp.int32, (N_TOKENS, N_EXPERTS), 1)
    onehot = (idxc == eid).astype(jnp.float32)
    TB = 128
    n_t = N_TOKENS // TB
    tril = (lax.broadcasted_iota(jnp.int32, (TB, TB), 0)
            >= lax.broadcasted_iota(jnp.int32, (TB, TB), 1)
            ).astype(jnp.float32)
    ones_row = jnp.ones((1, TB), jnp.float32)
    tots = jnp.concatenate(
        [jnp.dot(ones_row, onehot[TB * b:TB * (b + 1), :],
                 preferred_element_type=jnp.float32) for b in range(n_t)],
        axis=0)
    stri = (lax.broadcasted_iota(jnp.int32, (n_t, n_t), 0)
            > lax.broadcasted_iota(jnp.int32, (n_t, n_t), 1)
            ).astype(jnp.float32)
    carry = jnp.dot(stri, tots, preferred_element_type=jnp.float32)
    csum = jnp.concatenate(
        [jnp.dot(tril, onehot[TB * b:TB * (b + 1), :],
                 preferred_element_type=jnp.float32) + carry[b:b + 1, :]
         for b in range(n_t)], axis=0).astype(jnp.int32)
    onehot = onehot.astype(jnp.int32)
    rank = jnp.sum(onehot * csum, axis=1) - 1
    counts = csum[N_TOKENS - 1:N_TOKENS, :]
    nblocks = (counts + BR - 1) // BR
    triu = (lax.broadcasted_iota(jnp.int32, (N_EXPERTS, N_EXPERTS), 0)
            <= lax.broadcasted_iota(jnp.int32, (N_EXPERTS, N_EXPERTS), 1)
            ).astype(jnp.float32)
    cumblk = jnp.dot(nblocks.astype(jnp.float32), triu,
                     preferred_element_type=jnp.float32).astype(jnp.int32)
    padded_off = (cumblk - nblocks) * BR
    inv = jnp.sum(onehot * padded_off, axis=1) + rank
    inv_ref[...] = inv[:, None]
    bid = lax.broadcasted_iota(jnp.int32, (NBLK, N_EXPERTS), 0)
    be = jnp.sum((bid >= cumblk).astype(jnp.int32), axis=1)
    be_ref[...] = jnp.minimum(be, N_EXPERTS - 1)[None, :]
    cumblk_ref[...] = cumblk


def _gmm_body(be_ref, nb_ref, x_ref, w_ref, b_ref, o_ref):
    i = pl.program_id(0)

    @pl.when(i < nb_ref[0])
    def _():
        o_ref[...] = (
            jnp.dot(x_ref[...], w_ref[0], preferred_element_type=jnp.float32)
            + b_ref[0]
        )


def _sc_row_scatter(rows, dst_idx, n_out_rows):
    b, d = rows.shape
    rows_per_w = b // _SC_WORKERS
    mesh = plsc.VectorSubcoreMesh(
        core_axis_name="c", subcore_axis_name="s",
        num_cores=_SC_CORES, num_subcores=_SC_SUBCORES)

    @functools.partial(
        pl.kernel, mesh=mesh,
        out_type=jax.ShapeDtypeStruct((n_out_rows, d), jnp.float32),
        scratch_types=[
            pltpu.VMEM((rows_per_w,), jnp.int32),
            pltpu.VMEM((rows_per_w, d), jnp.float32),
            pltpu.SemaphoreType.DMA,
        ],
    )
    def k(rows_hbm, idx_hbm, out_hbm, idx_v, rows_v, sem):
        wid = lax.axis_index("s") * _SC_CORES + lax.axis_index("c")
        base = wid * rows_per_w
        pltpu.sync_copy(idx_hbm.at[pl.ds(base, rows_per_w)], idx_v)
        pltpu.sync_copy(rows_hbm.at[pl.ds(base, rows_per_w)], rows_v)
        pltpu.async_copy(rows_v, out_hbm.at[idx_v], sem).wait()

    return k(rows, dst_idx)


def _sc_row_gather(table, idx, n_rows, chunk):
    _, d = table.shape
    rows_per_w = n_rows // _SC_WORKERS
    n_chunks = rows_per_w // chunk
    mesh = plsc.VectorSubcoreMesh(
        core_axis_name="c", subcore_axis_name="s",
        num_cores=_SC_CORES, num_subcores=_SC_SUBCORES)

    @functools.partial(
        pl.kernel, mesh=mesh,
        out_type=jax.ShapeDtypeStruct((n_rows, d), jnp.float32),
        scratch_types=[
            pltpu.VMEM((rows_per_w,), jnp.int32),
            pltpu.VMEM((chunk, d), jnp.float32),
            pltpu.SemaphoreType.DMA,
        ],
    )
    def k(table_hbm, idx_hbm, out_hbm, idx_v, rows_v, sem):
        wid = lax.axis_index("s") * _SC_CORES + lax.axis_index("c")
        base = wid * rows_per_w
        pltpu.sync_copy(idx_hbm.at[pl.ds(base, rows_per_w)], idx_v)
        for c in range(n_chunks):
            pltpu.async_copy(
                table_hbm.at[idx_v.at[pl.ds(c * chunk, chunk)]],
                rows_v, sem).wait()
            pltpu.sync_copy(rows_v, out_hbm.at[pl.ds(base + c * chunk, chunk)])

    return k(table, idx)


def kernel(x, W_router, b_router, W_adapt, b_adapt, W_param, b_param):
    n_tiles = N_TOKENS // TOK_TILE
    idx3, tool_probs, params = pl.pallas_call(
        _router_body,
        grid=(n_tiles,),
        in_specs=[
            pl.BlockSpec((TOK_TILE, D_MODEL), lambda i: (i, 0)),
            pl.BlockSpec((D_MODEL, N_EXPERTS), lambda i: (0, 0)),
            pl.BlockSpec((1, N_EXPERTS), lambda i: (0, 0)),
            pl.BlockSpec((D_MODEL, P_DIM), lambda i: (0, 0)),
            pl.BlockSpec((1, P_DIM), lambda i: (0, 0)),
        ],
        out_specs=[
            pl.BlockSpec((1, 1, TOK_TILE), lambda i: (i, 0, 0)),
            pl.BlockSpec((TOK_TILE, N_EXPERTS), lambda i: (i, 0)),
            pl.BlockSpec((TOK_TILE, P_DIM), lambda i: (i, 0)),
        ],
        out_shape=[
            jax.ShapeDtypeStruct((n_tiles, 1, TOK_TILE), jnp.int32),
            jax.ShapeDtypeStruct((N_TOKENS, N_EXPERTS), jnp.float32),
            jax.ShapeDtypeStruct((N_TOKENS, P_DIM), jnp.float32),
        ],
    )(x, W_router, b_router.reshape(1, N_EXPERTS),
      W_param, b_param.reshape(1, P_DIM))
    tool_idx = idx3.reshape(N_TOKENS)

    inv2, be2, cumblk2 = pl.pallas_call(
        _bookkeep_body,
        out_shape=[
            jax.ShapeDtypeStruct((N_TOKENS, 1), jnp.int32),
            jax.ShapeDtypeStruct((1, NBLK), jnp.int32),
            jax.ShapeDtypeStruct((1, N_EXPERTS), jnp.int32),
        ],
    )(tool_idx.reshape(N_TOKENS, 1))
    inv_pos = inv2.reshape(N_TOKENS)
    block_expert = be2.reshape(NBLK)
    nblk_live = cumblk2[0, N_EXPERTS - 1:N_EXPERTS]

    x_sorted = _sc_row_scatter(x, inv_pos, PAD_ROWS)

    y_sorted = pl.pallas_call(
        _gmm_body,
        grid_spec=pltpu.PrefetchScalarGridSpec(
            num_scalar_prefetch=2,
            grid=(NBLK,),
            in_specs=[
                pl.BlockSpec((BR, D_MODEL),
                             lambda i, be, nb: (jnp.minimum(i, nb[0] - 1), 0)),
                pl.BlockSpec((1, D_MODEL, D_MODEL),
                             lambda i, be, nb: (be[i], 0, 0)),
                pl.BlockSpec((1, 1, D_MODEL), lambda i, be, nb: (be[i], 0, 0)),
            ],
            out_specs=pl.BlockSpec((BR, D_MODEL), lambda i, be, nb: (i, 0)),
        ),
        out_shape=jax.ShapeDtypeStruct((PAD_ROWS, D_MODEL), jnp.float32),
    )(block_expert, nblk_live, x_sorted, W_adapt,
      b_adapt.reshape(N_EXPERTS, 1, D_MODEL))

    adapted = _sc_row_gather(y_sorted, inv_pos, N_TOKENS, chunk=64)

    return (tool_idx, tool_probs, adapted, params)
